# baseline (speedup 1.0000x reference)
import functools
import math

import jax
import jax.numpy as jnp
from jax import lax
from jax.experimental import pallas as pl
from jax.experimental.pallas import tpu as pltpu
from jax.experimental.pallas import tpu_sc as plsc

EMBED_DIM = 64
BATCH = 4096
HIST = 200
VOCAB = 1000000
NUM_CORES = 2
NUM_SUBCORES = 16
NW = NUM_CORES * NUM_SUBCORES
BW = BATCH // NW
SCALE = math.sqrt(EMBED_DIM)
L = 16
TBLK = 1024
NTBLK = -(-VOCAB // TBLK)

_mesh = plsc.VectorSubcoreMesh(core_axis_name="c", subcore_axis_name="s")


@functools.partial(
    pl.pallas_call,
    grid=(NTBLK,),
    in_specs=[pl.BlockSpec((EMBED_DIM, TBLK), lambda j: (0, j))],
    out_specs=pl.BlockSpec((TBLK, 128), lambda j: (j, 0)),
    out_shape=jax.ShapeDtypeStruct((VOCAB, 128), jnp.float32),
)
def _repack(tt_ref, out_ref):
    r = lax.broadcasted_iota(jnp.int32, (EMBED_DIM, EMBED_DIM), 0)
    c = lax.broadcasted_iota(jnp.int32, (EMBED_DIM, EMBED_DIM), 1)
    eye_s = jnp.where(r == c, jnp.float32(SCALE), jnp.float32(0.0))
    t = lax.dot_general(
        tt_ref[...], eye_s, (((0,), (0,)), ((), ())),
        preferred_element_type=jnp.float32)
    out_ref[:, 0:EMBED_DIM] = t


@functools.partial(
    pl.kernel,
    mesh=_mesh,
    compiler_params=pltpu.CompilerParams(
        use_tc_tiling_on_sc=False, needs_layout_passes=False),
    out_type=jax.ShapeDtypeStruct((HIST, 8, NW, 8, 128), jnp.float32),
    scratch_types=[
        pltpu.VMEM((HIST // 8, 8, BW), jnp.int32),
        pltpu.VMEM((2, BW, 128), jnp.float32),
        pltpu.VMEM((2, 8, 8, BW), jnp.float32),
        pltpu.SemaphoreType.DMA,
        pltpu.SemaphoreType.DMA,
        pltpu.SemaphoreType.DMA,
        pltpu.SemaphoreType.DMA,
    ],
)
def _emb_lookup(idx_hbm, table_hbm, out_hbm, idx_v, rows_v, outb_v,
                isem, gsem, osem0, osem1):
    wid = lax.axis_index("s") * NUM_CORES + lax.axis_index("c")
    osems = (osem0, osem1)

    pltpu.async_copy(idx_hbm.at[:, wid], idx_v, isem).wait()

    def gather_pair(h, buf):
        th = lax.div(h, 8)
        sh = lax.rem(h, 8)
        return pltpu.async_copy(
            table_hbm.at[idx_v.at[th, sh]], rows_v.at[buf], gsem)

    def start_gather(h, buf):
        gather_pair(h, buf)

    def wait_gather(h, buf):
        pltpu.make_async_copy(
            table_hbm.at[idx_v.at[lax.div(h, 8), lax.rem(h, 8)]],
            rows_v.at[buf], gsem,
        ).wait()

    def slab_compute(h, buf):
        rows = rows_v.at[buf]
        lanes = lax.iota(jnp.int32, L)
        for g in range(BW // L):
            rowi = lanes + g * L
            sl = pl.ds(g * L, L)
            for d in range(EMBED_DIM):
                vals = plsc.load_gather(
                    rows, [rowi, jnp.full((L,), d, jnp.int32)])
                outb_v[buf, d // 8, d % 8, sl] = vals

        pltpu.async_copy(
            outb_v.at[buf], out_hbm.at[h, pl.ds(0, 8), wid], osems[buf])

    def wait_out(h, buf):
        pltpu.make_async_copy(
            outb_v.at[buf], out_hbm.at[h, pl.ds(0, 8), wid], osems[buf]
        ).wait()

    start_gather(0, 0)
    start_gather(1, 1)

    def body(k, carry):
        h0 = 2 * k
        for buf in range(2):
            h = h0 + buf
            wait_gather(h, buf)

            @pl.when(k > 0)
            def _():
                wait_out(h, buf)

            slab_compute(h, buf)

            @pl.when(k < HIST // 2 - 1)
            def _():
                start_gather(h + 2, buf)

        return carry

    lax.fori_loop(0, HIST // 2, body, 0, unroll=False)
    wait_out(HIST - 2, 0)
    wait_out(HIST - 1, 1)


def kernel(input, table):
    idx4 = jnp.transpose(
        input.reshape(NW, BW, HIST // 8, 8), (2, 0, 3, 1))
    table_rm = _repack(jnp.transpose(table))
    out5 = _emb_lookup(idx4, table_rm)
    return jnp.transpose(out5, (2, 4, 0, 1, 3)).reshape(BATCH, HIST, EMBED_DIM)


# speedup vs baseline: 2.0609x; 2.0609x over previous
import functools
import math

import jax
import jax.numpy as jnp
from jax import lax
from jax.experimental import pallas as pl
from jax.experimental.pallas import tpu as pltpu
from jax.experimental.pallas import tpu_sc as plsc

EMBED_DIM = 64
BATCH = 4096
HIST = 200
VOCAB = 1000000
NUM_CORES = 2
NUM_SUBCORES = 16
NW = NUM_CORES * NUM_SUBCORES
BW = BATCH // NW
SCALE = math.sqrt(EMBED_DIM)
L = 16
TBLK = 4096
NTBLK = -(-VOCAB // TBLK)

_mesh = plsc.VectorSubcoreMesh(core_axis_name="c", subcore_axis_name="s")


@functools.partial(
    pl.pallas_call,
    grid=(NTBLK,),
    in_specs=[pl.BlockSpec((EMBED_DIM, TBLK), lambda j: (0, j))],
    out_specs=pl.BlockSpec((TBLK, 128), lambda j: (j, 0)),
    out_shape=jax.ShapeDtypeStruct((VOCAB, 128), jnp.float32),
)
def _repack(tt_ref, out_ref):
    t = jnp.transpose(tt_ref[...]) * SCALE
    out_ref[...] = jnp.concatenate([t, t], axis=1)


@functools.partial(
    pl.kernel,
    mesh=_mesh,
    compiler_params=pltpu.CompilerParams(
        use_tc_tiling_on_sc=False, needs_layout_passes=False),
    out_type=jax.ShapeDtypeStruct((HIST, 8, NW, 8, 128), jnp.float32),
    scratch_types=[
        pltpu.VMEM((HIST // 8, 8, BW), jnp.int32),
        pltpu.VMEM((2, BW, 128), jnp.float32),
        pltpu.VMEM((2, 8, 8, BW), jnp.float32),
        pltpu.SemaphoreType.DMA,
        pltpu.SemaphoreType.DMA,
        pltpu.SemaphoreType.DMA,
        pltpu.SemaphoreType.DMA,
    ],
)
def _emb_lookup(idx_hbm, table_hbm, out_hbm, idx_v, rows_v, outb_v,
                isem, gsem, osem0, osem1):
    wid = lax.axis_index("s") * NUM_CORES + lax.axis_index("c")
    osems = (osem0, osem1)

    pltpu.async_copy(idx_hbm.at[:, wid], idx_v, isem).wait()

    def gather_pair(h, buf):
        th = lax.div(h, 8)
        sh = lax.rem(h, 8)
        return pltpu.async_copy(
            table_hbm.at[idx_v.at[th, sh]], rows_v.at[buf], gsem)

    def start_gather(h, buf):
        gather_pair(h, buf)

    def wait_gather(h, buf):
        pltpu.make_async_copy(
            table_hbm.at[idx_v.at[lax.div(h, 8), lax.rem(h, 8)]],
            rows_v.at[buf], gsem,
        ).wait()

    def slab_compute(h, buf):
        rows = rows_v.at[buf]
        lanes = lax.iota(jnp.int32, L)
        zeros = jnp.zeros((L,), jnp.int32)
        for g in range(BW // L):
            rowi = lanes + g * L
            sl = pl.ds(g * L, L)

            @plsc.parallel_loop(0, EMBED_DIM, 1, unroll=8)
            def _(d):
                vals = plsc.load_gather(rows, [rowi, zeros + d])
                dt = lax.shift_right_logical(d, 3)
                dl = jnp.bitwise_and(d, 7)
                outb_v[buf, dt, dl, sl] = vals

        pltpu.async_copy(
            outb_v.at[buf], out_hbm.at[h, pl.ds(0, 8), wid], osems[buf])

    def wait_out(h, buf):
        pltpu.make_async_copy(
            outb_v.at[buf], out_hbm.at[h, pl.ds(0, 8), wid], osems[buf]
        ).wait()

    start_gather(0, 0)
    start_gather(1, 1)

    def body(k, carry):
        h0 = 2 * k
        for buf in range(2):
            h = h0 + buf
            wait_gather(h, buf)

            @pl.when(k > 0)
            def _():
                wait_out(h, buf)

            slab_compute(h, buf)

            @pl.when(k < HIST // 2 - 1)
            def _():
                start_gather(h + 2, buf)

        return carry

    lax.fori_loop(0, HIST // 2, body, 0, unroll=False)
    wait_out(HIST - 2, 0)
    wait_out(HIST - 1, 1)


def kernel(input, table):
    idx4 = jnp.transpose(
        input.reshape(NW, BW, HIST // 8, 8), (2, 0, 3, 1))
    table_rm = _repack(jnp.transpose(table))
    out5 = _emb_lookup(idx4, table_rm)
    return jnp.transpose(out5, (2, 4, 0, 1, 3)).reshape(BATCH, HIST, EMBED_DIM)


# speedup vs baseline: 3.1560x; 1.5314x over previous
import functools
import math

import jax
import jax.numpy as jnp
from jax import lax
from jax.experimental import pallas as pl
from jax.experimental.pallas import tpu as pltpu
from jax.experimental.pallas import tpu_sc as plsc

EMBED_DIM = 64
BATCH = 4096
HIST = 200
VOCAB = 1000000
NUM_CORES = 2
NUM_SUBCORES = 16
NW = NUM_CORES * NUM_SUBCORES
BW = BATCH // NW
SCALE = math.sqrt(EMBED_DIM)
L = 16
TBLK = 4096
NTBLK = -(-VOCAB // TBLK)

_mesh = plsc.VectorSubcoreMesh(core_axis_name="c", subcore_axis_name="s")


@functools.partial(
    pl.pallas_call,
    grid=(NTBLK,),
    in_specs=[pl.BlockSpec((EMBED_DIM, TBLK), lambda j: (0, j))],
    out_specs=pl.BlockSpec((TBLK, 128), lambda j: (j, 0)),
    out_shape=jax.ShapeDtypeStruct((VOCAB, 128), jnp.float32),
)
def _repack(tt_ref, out_ref):
    t = jnp.transpose(tt_ref[...]) * SCALE
    out_ref[...] = jnp.concatenate([t, t], axis=1)


@functools.partial(
    pl.kernel,
    mesh=_mesh,
    compiler_params=pltpu.CompilerParams(
        use_tc_tiling_on_sc=False, needs_layout_passes=False),
    out_type=jax.ShapeDtypeStruct((HIST, 8, NW, 8, 128), jnp.float32),
    scratch_types=[
        pltpu.VMEM((HIST // 8, 8, BW), jnp.int32),
        pltpu.VMEM((2, BW, 128), jnp.float32),
        pltpu.VMEM((2, 8, 8, BW), jnp.float32),
        pltpu.SemaphoreType.DMA,
        pltpu.SemaphoreType.DMA,
        pltpu.SemaphoreType.DMA,
        pltpu.SemaphoreType.DMA,
    ],
)
def _emb_lookup(idx_hbm, table_hbm, out_hbm, idx_v, rows_v, outb_v,
                isem, gsem, osem0, osem1):
    wid = lax.axis_index("s") * NUM_CORES + lax.axis_index("c")
    osems = (osem0, osem1)

    pltpu.async_copy(idx_hbm.at[:, wid], idx_v, isem).wait()

    def gather_pair(h, buf):
        th = lax.div(h, 8)
        sh = lax.rem(h, 8)
        return pltpu.async_copy(
            table_hbm.at[idx_v.at[th, sh]], rows_v.at[buf], gsem)

    def start_gather(h, buf):
        gather_pair(h, buf)

    def wait_gather(h, buf):
        pltpu.make_async_copy(
            table_hbm.at[idx_v.at[lax.div(h, 8), lax.rem(h, 8)]],
            rows_v.at[buf], gsem,
        ).wait()

    def slab_compute(h, buf):
        rows = rows_v.at[buf]
        outb = outb_v.at[buf]
        lanes = lax.iota(jnp.int32, L)
        for g in range(BW // L):
            bvec = lanes + g * L

            @plsc.parallel_loop(0, EMBED_DIM, 1, unroll=8)
            def _(j):
                k = jnp.bitwise_and(j, L - 1)
                rot = jnp.bitwise_and(lanes + k, L - 1)
                dvec = jnp.bitwise_and(j, 0x30) + rot
                vals = plsc.load_gather(rows, [bvec, dvec])
                dt = lax.shift_right_logical(dvec, 3)
                dl = jnp.bitwise_and(dvec, 7)
                plsc.store_scatter(outb, [dt, dl, bvec], vals)

        pltpu.async_copy(
            outb_v.at[buf], out_hbm.at[h, pl.ds(0, 8), wid], osems[buf])

    def wait_out(h, buf):
        pltpu.make_async_copy(
            outb_v.at[buf], out_hbm.at[h, pl.ds(0, 8), wid], osems[buf]
        ).wait()

    start_gather(0, 0)
    start_gather(1, 1)

    def body(k, carry):
        h0 = 2 * k
        for buf in range(2):
            h = h0 + buf
            wait_gather(h, buf)

            @pl.when(k > 0)
            def _():
                wait_out(h, buf)

            slab_compute(h, buf)

            @pl.when(k < HIST // 2 - 1)
            def _():
                start_gather(h + 2, buf)

        return carry

    lax.fori_loop(0, HIST // 2, body, 0, unroll=False)
    wait_out(HIST - 2, 0)
    wait_out(HIST - 1, 1)


def kernel(input, table):
    idx4 = jnp.transpose(
        input.reshape(NW, BW, HIST // 8, 8), (2, 0, 3, 1))
    table_rm = _repack(jnp.transpose(table))
    out5 = _emb_lookup(idx4, table_rm)
    return jnp.transpose(out5, (2, 4, 0, 1, 3)).reshape(BATCH, HIST, EMBED_DIM)


# speedup vs baseline: 3.7632x; 1.1924x over previous
import functools
import math

import jax
import jax.numpy as jnp
from jax import lax
from jax.experimental import pallas as pl
from jax.experimental.pallas import tpu as pltpu
from jax.experimental.pallas import tpu_sc as plsc

EMBED_DIM = 64
BATCH = 4096
HIST = 200
VOCAB = 1000000
NUM_CORES = 2
NUM_SUBCORES = 16
NW = NUM_CORES * NUM_SUBCORES
BW = BATCH // NW
SCALE = math.sqrt(EMBED_DIM)
L = 16
TBLK = 8192
NTBLK = -(-VOCAB // TBLK)

_mesh = plsc.VectorSubcoreMesh(core_axis_name="c", subcore_axis_name="s")


@functools.partial(
    pl.pallas_call,
    grid=(NTBLK,),
    in_specs=[pl.BlockSpec((EMBED_DIM, TBLK), lambda j: (0, j))],
    out_specs=pl.BlockSpec((TBLK, 128), lambda j: (j, 0)),
    out_shape=jax.ShapeDtypeStruct((VOCAB, 128), jnp.float32),
)
def _repack(tt_ref, out_ref):
    out_ref[:, 0:EMBED_DIM] = jnp.transpose(tt_ref[...])


@functools.partial(
    pl.kernel,
    mesh=_mesh,
    compiler_params=pltpu.CompilerParams(
        use_tc_tiling_on_sc=False, needs_layout_passes=False),
    out_type=jax.ShapeDtypeStruct((HIST, 8, NW, 8, 128), jnp.float32),
    scratch_types=[
        pltpu.VMEM((HIST // 8, 8, BW), jnp.int32),
        pltpu.VMEM((2, BW, 128), jnp.float32),
        pltpu.VMEM((2, 8, 8, BW), jnp.float32),
        pltpu.SemaphoreType.DMA,
        pltpu.SemaphoreType.DMA,
        pltpu.SemaphoreType.DMA,
        pltpu.SemaphoreType.DMA,
    ],
)
def _emb_lookup(idx_hbm, table_hbm, out_hbm, idx_v, rows_v, outb_v,
                isem, gsem, osem0, osem1):
    wid = lax.axis_index("s") * NUM_CORES + lax.axis_index("c")
    osems = (osem0, osem1)

    pltpu.async_copy(idx_hbm.at[:, wid], idx_v, isem).wait()

    def gather_pair(h, buf):
        th = lax.div(h, 8)
        sh = lax.rem(h, 8)
        return pltpu.async_copy(
            table_hbm.at[idx_v.at[th, sh]], rows_v.at[buf], gsem)

    def start_gather(h, buf):
        gather_pair(h, buf)

    def wait_gather(h, buf):
        pltpu.make_async_copy(
            table_hbm.at[idx_v.at[lax.div(h, 8), lax.rem(h, 8)]],
            rows_v.at[buf], gsem,
        ).wait()

    def slab_compute(h, buf):
        rows = rows_v.at[buf]
        outb = outb_v.at[buf]
        lanes = lax.iota(jnp.int32, L)
        for g in range(BW // L):
            bvec = lanes + g * L

            @plsc.parallel_loop(0, EMBED_DIM, 1, unroll=8)
            def _(j):
                k = jnp.bitwise_and(j, L - 1)
                rot = jnp.bitwise_and(lanes + k, L - 1)
                dvec = jnp.bitwise_and(j, 0x30) + rot
                vals = plsc.load_gather(rows, [bvec, dvec])
                dt = lax.shift_right_logical(dvec, 3)
                dl = jnp.bitwise_and(dvec, 7)
                plsc.store_scatter(outb, [dt, dl, bvec], vals * SCALE)

        pltpu.async_copy(
            outb_v.at[buf], out_hbm.at[h, pl.ds(0, 8), wid], osems[buf])

    def wait_out(h, buf):
        pltpu.make_async_copy(
            outb_v.at[buf], out_hbm.at[h, pl.ds(0, 8), wid], osems[buf]
        ).wait()

    start_gather(0, 0)
    start_gather(1, 1)

    def body(k, carry):
        h0 = 2 * k
        for buf in range(2):
            h = h0 + buf
            wait_gather(h, buf)

            @pl.when(k > 0)
            def _():
                wait_out(h, buf)

            slab_compute(h, buf)

            @pl.when(k < HIST // 2 - 1)
            def _():
                start_gather(h + 2, buf)

        return carry

    lax.fori_loop(0, HIST // 2, body, 0, unroll=False)
    wait_out(HIST - 2, 0)
    wait_out(HIST - 1, 1)


def kernel(input, table):
    idx4 = jnp.transpose(
        input.reshape(NW, BW, HIST // 8, 8), (2, 0, 3, 1))
    table_rm = _repack(jnp.transpose(table))
    out5 = _emb_lookup(idx4, table_rm)
    return jnp.transpose(out5, (2, 4, 0, 1, 3)).reshape(BATCH, HIST, EMBED_DIM)


# speedup vs baseline: 3.9051x; 1.0377x over previous
import functools
import math

import jax
import jax.numpy as jnp
from jax import lax
from jax.experimental import pallas as pl
from jax.experimental.pallas import tpu as pltpu
from jax.experimental.pallas import tpu_sc as plsc

EMBED_DIM = 64
BATCH = 4096
HIST = 200
VOCAB = 1000000
NUM_CORES = 2
NUM_SUBCORES = 16
NW = NUM_CORES * NUM_SUBCORES
BW = BATCH // NW
SCALE = math.sqrt(EMBED_DIM)
L = 16
TBLK = 8192
NTBLK = -(-VOCAB // TBLK)

_mesh = plsc.VectorSubcoreMesh(core_axis_name="c", subcore_axis_name="s")


@functools.partial(
    pl.pallas_call,
    grid=(NTBLK,),
    in_specs=[pl.BlockSpec((EMBED_DIM, TBLK), lambda j: (0, j))],
    out_specs=pl.BlockSpec((TBLK, 128), lambda j: (j, 0)),
    out_shape=jax.ShapeDtypeStruct((VOCAB, 128), jnp.float32),
)
def _repack(tt_ref, out_ref):
    out_ref[:, 0:EMBED_DIM] = jnp.transpose(tt_ref[...])


@functools.partial(
    pl.kernel,
    mesh=_mesh,
    compiler_params=pltpu.CompilerParams(
        use_tc_tiling_on_sc=False, needs_layout_passes=False),
    out_type=jax.ShapeDtypeStruct((HIST, 8, NW, 8, 128), jnp.float32),
    scratch_types=[
        pltpu.VMEM((HIST // 8, 8, BW), jnp.int32),
        pltpu.VMEM((2, 4 * BW), jnp.int32),
        pltpu.VMEM((2, 4 * BW, L), jnp.float32),
        pltpu.VMEM((2, 8, 8, BW), jnp.float32),
        pltpu.SemaphoreType.DMA,
        pltpu.SemaphoreType.DMA,
        pltpu.SemaphoreType.DMA,
        pltpu.SemaphoreType.DMA,
    ],
)
def _emb_lookup(idx_hbm, table_hbm, out_hbm, idx_v, idx4_v, rows_v, outb_v,
                isem, gsem, osem0, osem1):
    wid = lax.axis_index("s") * NUM_CORES + lax.axis_index("c")
    osems = (osem0, osem1)

    pltpu.async_copy(idx_hbm.at[:, wid], idx_v, isem).wait()

    lanes0 = lax.iota(jnp.int32, L)
    quarter = lax.shift_right_logical(lanes0, 2)
    low2 = jnp.bitwise_and(lanes0, 3)

    def start_gather(h, buf):
        islab = idx_v.at[lax.div(h, 8), lax.rem(h, 8)]

        @plsc.parallel_loop(0, 4 * BW // L, 1, unroll=8)
        def _(m):
            v = plsc.load_gather(islab, [4 * m + quarter])
            idx4_v[buf, pl.ds(m * L, L)] = lax.shift_left(v, 3) + low2

        for c in range(4):
            pltpu.async_copy(
                table_hbm.at[idx4_v.at[buf, pl.ds(c * BW, BW)]],
                rows_v.at[buf, pl.ds(c * BW, BW)], gsem)

    def wait_gather(h, buf):
        for c in range(4):
            pltpu.make_async_copy(
                table_hbm.at[idx4_v.at[buf, pl.ds(c * BW, BW)]],
                rows_v.at[buf, pl.ds(c * BW, BW)], gsem,
            ).wait()

    def slab_compute(h, buf):
        rows = rows_v.at[buf]
        outb = outb_v.at[buf]
        lanes = lax.iota(jnp.int32, L)
        for g in range(BW // L):
            bvec = lanes + g * L

            @plsc.parallel_loop(0, EMBED_DIM, 1, unroll=8)
            def _(j):
                k = jnp.bitwise_and(j, L - 1)
                rot = jnp.bitwise_and(lanes + k, L - 1)
                dvec = jnp.bitwise_and(j, 0x30) + rot
                ri = 4 * bvec + lax.shift_right_logical(dvec, 4)
                vals = plsc.load_gather(rows, [ri, jnp.bitwise_and(dvec, 15)])
                dt = lax.shift_right_logical(dvec, 3)
                dl = jnp.bitwise_and(dvec, 7)
                plsc.store_scatter(outb, [dt, dl, bvec], vals * SCALE)

        pltpu.async_copy(
            outb_v.at[buf], out_hbm.at[h, pl.ds(0, 8), wid], osems[buf])

    def wait_out(h, buf):
        pltpu.make_async_copy(
            outb_v.at[buf], out_hbm.at[h, pl.ds(0, 8), wid], osems[buf]
        ).wait()

    start_gather(0, 0)
    start_gather(1, 1)

    def body(k, carry):
        h0 = 2 * k
        for buf in range(2):
            h = h0 + buf
            wait_gather(h, buf)

            @pl.when(k > 0)
            def _():
                wait_out(h, buf)

            slab_compute(h, buf)

            @pl.when(k < HIST // 2 - 1)
            def _():
                start_gather(h + 2, buf)

        return carry

    lax.fori_loop(0, HIST // 2, body, 0, unroll=False)
    wait_out(HIST - 2, 0)
    wait_out(HIST - 1, 1)


def kernel(input, table):
    idx4 = jnp.transpose(
        input.reshape(NW, BW, HIST // 8, 8), (2, 0, 3, 1))
    table_rm = _repack(jnp.transpose(table)).reshape(8 * VOCAB, L)
    out5 = _emb_lookup(idx4, table_rm)
    return jnp.transpose(out5, (2, 4, 0, 1, 3)).reshape(BATCH, HIST, EMBED_DIM)


# speedup vs baseline: 4.0417x; 1.0350x over previous
import functools
import math

import jax
import jax.numpy as jnp
from jax import lax
from jax.experimental import pallas as pl
from jax.experimental.pallas import tpu as pltpu
from jax.experimental.pallas import tpu_sc as plsc

EMBED_DIM = 64
BATCH = 4096
HIST = 200
VOCAB = 1000000
NUM_CORES = 2
NUM_SUBCORES = 16
NW = NUM_CORES * NUM_SUBCORES
BW = BATCH // NW
SCALE = math.sqrt(EMBED_DIM)
L = 16
TBLK = 16384
NTBLK = -(-VOCAB // TBLK)

_mesh = plsc.VectorSubcoreMesh(core_axis_name="c", subcore_axis_name="s")


@functools.partial(
    pl.pallas_call,
    grid=(NTBLK,),
    in_specs=[pl.BlockSpec((EMBED_DIM, TBLK), lambda j: (0, j))],
    out_specs=pl.BlockSpec((TBLK, 128), lambda j: (j, 0)),
    out_shape=jax.ShapeDtypeStruct((VOCAB, 128), jnp.float32),
)
def _repack(tt_ref, out_ref):
    out_ref[:, 0:EMBED_DIM] = jnp.transpose(tt_ref[...])


@functools.partial(
    pl.kernel,
    mesh=_mesh,
    compiler_params=pltpu.CompilerParams(
        use_tc_tiling_on_sc=False, needs_layout_passes=False),
    out_type=jax.ShapeDtypeStruct((HIST, 8, NW, 8, 128), jnp.float32),
    scratch_types=[
        pltpu.VMEM((HIST // 8, 8, BW), jnp.int32),
        pltpu.VMEM((2, 4 * BW), jnp.int32),
        pltpu.VMEM((2, 4 * BW, L), jnp.float32),
        pltpu.VMEM((2, 8, 8, BW), jnp.float32),
        pltpu.SemaphoreType.DMA,
        pltpu.SemaphoreType.DMA,
        pltpu.SemaphoreType.DMA,
        pltpu.SemaphoreType.DMA,
    ],
)
def _emb_lookup(idx_hbm, table_hbm, out_hbm, idx_v, idx4_v, rows_v, outb_v,
                isem, gsem, osem0, osem1):
    wid = lax.axis_index("s") * NUM_CORES + lax.axis_index("c")
    osems = (osem0, osem1)

    pltpu.async_copy(idx_hbm.at[:, wid], idx_v, isem).wait()

    lanes0 = lax.iota(jnp.int32, L)
    quarter = lax.shift_right_logical(lanes0, 2)
    low2 = jnp.bitwise_and(lanes0, 3)

    def start_gather(h, buf):
        islab = idx_v.at[lax.div(h, 8), lax.rem(h, 8)]

        @plsc.parallel_loop(0, 4 * BW // L, 1, unroll=8)
        def _(m):
            v = plsc.load_gather(islab, [4 * m + quarter])
            idx4_v[buf, pl.ds(m * L, L)] = lax.shift_left(v, 3) + low2

        for c in range(4):
            pltpu.async_copy(
                table_hbm.at[idx4_v.at[buf, pl.ds(c * BW, BW)]],
                rows_v.at[buf, pl.ds(c * BW, BW)], gsem)

    def wait_gather(h, buf):
        for c in range(4):
            pltpu.make_async_copy(
                table_hbm.at[idx4_v.at[buf, pl.ds(c * BW, BW)]],
                rows_v.at[buf, pl.ds(c * BW, BW)], gsem,
            ).wait()

    def slab_compute(h, buf):
        rows = rows_v.at[buf]
        outb = outb_v.at[buf]
        lanes = lax.iota(jnp.int32, L)
        for g in range(BW // L):
            bvec = lanes + g * L

            @plsc.parallel_loop(0, EMBED_DIM, 1, unroll=8)
            def _(j):
                k = jnp.bitwise_and(j, L - 1)
                rot = jnp.bitwise_and(lanes + k, L - 1)
                dvec = jnp.bitwise_and(j, 0x30) + rot
                ri = 4 * bvec + lax.shift_right_logical(dvec, 4)
                vals = plsc.load_gather(rows, [ri, jnp.bitwise_and(dvec, 15)])
                dt = lax.shift_right_logical(dvec, 3)
                dl = jnp.bitwise_and(dvec, 7)
                plsc.store_scatter(outb, [dt, dl, bvec], vals * SCALE)

        pltpu.async_copy(
            outb_v.at[buf], out_hbm.at[h, pl.ds(0, 8), wid], osems[buf])

    def wait_out(h, buf):
        pltpu.make_async_copy(
            outb_v.at[buf], out_hbm.at[h, pl.ds(0, 8), wid], osems[buf]
        ).wait()

    start_gather(0, 0)
    start_gather(1, 1)

    def body(k, carry):
        h0 = 2 * k
        for buf in range(2):
            h = h0 + buf
            wait_gather(h, buf)

            @pl.when(k > 0)
            def _():
                wait_out(h, buf)

            slab_compute(h, buf)

            @pl.when(k < HIST // 2 - 1)
            def _():
                start_gather(h + 2, buf)

        return carry

    lax.fori_loop(0, HIST // 2, body, 0, unroll=False)
    wait_out(HIST - 2, 0)
    wait_out(HIST - 1, 1)


def kernel(input, table):
    idx4 = jnp.transpose(
        input.reshape(NW, BW, HIST // 8, 8), (2, 0, 3, 1))
    table_rm = _repack(jnp.transpose(table)).reshape(8 * VOCAB, L)
    out5 = _emb_lookup(idx4, table_rm)
    return jnp.transpose(out5, (2, 4, 0, 1, 3)).reshape(BATCH, HIST, EMBED_DIM)


# speedup vs baseline: 4.0955x; 1.0133x over previous
import functools
import math

import jax
import jax.numpy as jnp
from jax import lax
from jax.experimental import pallas as pl
from jax.experimental.pallas import tpu as pltpu
from jax.experimental.pallas import tpu_sc as plsc

EMBED_DIM = 64
BATCH = 4096
HIST = 200
VOCAB = 1000000
NUM_CORES = 2
NUM_SUBCORES = 16
NW = NUM_CORES * NUM_SUBCORES
BW = BATCH // NW
SCALE = math.sqrt(EMBED_DIM)
L = 16
TBLK = 32768
NTBLK = -(-VOCAB // TBLK)

_mesh = plsc.VectorSubcoreMesh(core_axis_name="c", subcore_axis_name="s")


@functools.partial(
    pl.pallas_call,
    grid=(NTBLK,),
    in_specs=[pl.BlockSpec((EMBED_DIM, TBLK), lambda j: (0, j))],
    out_specs=pl.BlockSpec((TBLK, 128), lambda j: (j, 0)),
    out_shape=jax.ShapeDtypeStruct((VOCAB, 128), jnp.float32),
)
def _repack(tt_ref, out_ref):
    out_ref[:, 0:EMBED_DIM] = jnp.transpose(tt_ref[...])


@functools.partial(
    pl.kernel,
    mesh=_mesh,
    compiler_params=pltpu.CompilerParams(
        use_tc_tiling_on_sc=False, needs_layout_passes=False),
    out_type=jax.ShapeDtypeStruct((HIST, 8, NW, 8, 128), jnp.float32),
    scratch_types=[
        pltpu.VMEM((HIST // 8, 8, BW), jnp.int32),
        pltpu.VMEM((2, 4 * BW), jnp.int32),
        pltpu.VMEM((2, 4 * BW, L), jnp.float32),
        pltpu.VMEM((2, 8, 8, BW), jnp.float32),
        pltpu.SemaphoreType.DMA,
        pltpu.SemaphoreType.DMA,
        pltpu.SemaphoreType.DMA,
        pltpu.SemaphoreType.DMA,
    ],
)
def _emb_lookup(idx_hbm, table_hbm, out_hbm, idx_v, idx4_v, rows_v, outb_v,
                isem, gsem, osem0, osem1):
    wid = lax.axis_index("s") * NUM_CORES + lax.axis_index("c")
    osems = (osem0, osem1)

    pltpu.async_copy(idx_hbm.at[:, wid], idx_v, isem).wait()

    lanes0 = lax.iota(jnp.int32, L)
    quarter = lax.shift_right_logical(lanes0, 2)
    low2 = jnp.bitwise_and(lanes0, 3)

    def start_gather(h, buf):
        islab = idx_v.at[lax.div(h, 8), lax.rem(h, 8)]

        @plsc.parallel_loop(0, 4 * BW // L, 1, unroll=8)
        def _(m):
            v = plsc.load_gather(islab, [4 * m + quarter])
            idx4_v[buf, pl.ds(m * L, L)] = lax.shift_left(v, 3) + low2

        for c in range(4):
            pltpu.async_copy(
                table_hbm.at[idx4_v.at[buf, pl.ds(c * BW, BW)]],
                rows_v.at[buf, pl.ds(c * BW, BW)], gsem)

    def wait_gather(h, buf):
        for c in range(4):
            pltpu.make_async_copy(
                table_hbm.at[idx4_v.at[buf, pl.ds(c * BW, BW)]],
                rows_v.at[buf, pl.ds(c * BW, BW)], gsem,
            ).wait()

    def slab_compute(h, buf):
        rows = rows_v.at[buf]
        outb = outb_v.at[buf]
        lanes = lax.iota(jnp.int32, L)
        for g in range(BW // L):
            bvec = lanes + g * L

            @plsc.parallel_loop(0, EMBED_DIM, 1, unroll=8)
            def _(j):
                k = jnp.bitwise_and(j, L - 1)
                rot = jnp.bitwise_and(lanes + k, L - 1)
                dvec = jnp.bitwise_and(j, 0x30) + rot
                ri = 4 * bvec + lax.shift_right_logical(dvec, 4)
                vals = plsc.load_gather(rows, [ri, jnp.bitwise_and(dvec, 15)])
                dt = lax.shift_right_logical(dvec, 3)
                dl = jnp.bitwise_and(dvec, 7)
                plsc.store_scatter(outb, [dt, dl, bvec], vals * SCALE)

        pltpu.async_copy(
            outb_v.at[buf], out_hbm.at[h, pl.ds(0, 8), wid], osems[buf])

    def wait_out(h, buf):
        pltpu.make_async_copy(
            outb_v.at[buf], out_hbm.at[h, pl.ds(0, 8), wid], osems[buf]
        ).wait()

    start_gather(0, 0)
    start_gather(1, 1)

    def body(k, carry):
        h0 = 2 * k
        for buf in range(2):
            h = h0 + buf
            wait_gather(h, buf)

            @pl.when(k > 0)
            def _():
                wait_out(h, buf)

            slab_compute(h, buf)

            @pl.when(k < HIST // 2 - 1)
            def _():
                start_gather(h + 2, buf)

        return carry

    lax.fori_loop(0, HIST // 2, body, 0, unroll=False)
    wait_out(HIST - 2, 0)
    wait_out(HIST - 1, 1)


def kernel(input, table):
    idx4 = jnp.transpose(
        input.reshape(NW, BW, HIST // 8, 8), (2, 0, 3, 1))
    table_rm = _repack(jnp.transpose(table)).reshape(8 * VOCAB, L)
    out5 = _emb_lookup(idx4, table_rm)
    return jnp.transpose(out5, (2, 4, 0, 1, 3)).reshape(BATCH, HIST, EMBED_DIM)


# speedup vs baseline: 4.3724x; 1.0676x over previous
import functools
import math

import jax
import jax.numpy as jnp
from jax import lax
from jax.experimental import pallas as pl
from jax.experimental.pallas import tpu as pltpu
from jax.experimental.pallas import tpu_sc as plsc

EMBED_DIM = 64
BATCH = 4096
HIST = 200
VOCAB = 1000000
NUM_CORES = 2
NUM_SUBCORES = 16
NW = NUM_CORES * NUM_SUBCORES
BW = BATCH // NW
SCALE = math.sqrt(EMBED_DIM)
L = 16
TBLK = 32768
NTBLK = -(-VOCAB // TBLK)

_mesh = plsc.VectorSubcoreMesh(core_axis_name="c", subcore_axis_name="s")


@functools.partial(
    pl.pallas_call,
    grid=(NTBLK,),
    in_specs=[pl.BlockSpec((EMBED_DIM, TBLK), lambda j: (0, j))],
    out_specs=pl.BlockSpec((TBLK, 128), lambda j: (j, 0)),
    out_shape=jax.ShapeDtypeStruct((VOCAB, 128), jnp.float32),
)
def _repack(tt_ref, out_ref):
    out_ref[:, 0:EMBED_DIM] = jnp.transpose(tt_ref[...])


@functools.partial(
    pl.kernel,
    mesh=_mesh,
    compiler_params=pltpu.CompilerParams(
        use_tc_tiling_on_sc=False, needs_layout_passes=False),
    out_type=jax.ShapeDtypeStruct((HIST, 8, NW, 8, 128), jnp.float32),
    scratch_types=[
        pltpu.VMEM((HIST // 8, 8, BW), jnp.int32),
        pltpu.VMEM((4, 4 * BW), jnp.int32),
        pltpu.VMEM((4, 4 * BW, L), jnp.float32),
        pltpu.VMEM((4, 8, 8, BW), jnp.float32),
        pltpu.SemaphoreType.DMA,
        pltpu.SemaphoreType.DMA,
        pltpu.SemaphoreType.DMA,
        pltpu.SemaphoreType.DMA,
        pltpu.SemaphoreType.DMA,
        pltpu.SemaphoreType.DMA,
    ],
)
def _emb_lookup(idx_hbm, table_hbm, out_hbm, idx_v, idx4_v, rows_v, outb_v,
                isem, gsem, osem0, osem1, osem2, osem3):
    wid = lax.axis_index("s") * NUM_CORES + lax.axis_index("c")
    osems = (osem0, osem1, osem2, osem3)

    pltpu.async_copy(idx_hbm.at[:, wid], idx_v, isem).wait()

    lanes0 = lax.iota(jnp.int32, L)
    quarter = lax.shift_right_logical(lanes0, 2)
    low2 = jnp.bitwise_and(lanes0, 3)

    def start_gather(h, buf):
        islab = idx_v.at[lax.div(h, 8), lax.rem(h, 8)]

        @plsc.parallel_loop(0, 4 * BW // L, 1, unroll=8)
        def _(m):
            v = plsc.load_gather(islab, [4 * m + quarter])
            idx4_v[buf, pl.ds(m * L, L)] = lax.shift_left(v, 3) + low2

        for c in range(4):
            pltpu.async_copy(
                table_hbm.at[idx4_v.at[buf, pl.ds(c * BW, BW)]],
                rows_v.at[buf, pl.ds(c * BW, BW)], gsem)

    def wait_gather(h, buf):
        for c in range(4):
            pltpu.make_async_copy(
                table_hbm.at[idx4_v.at[buf, pl.ds(c * BW, BW)]],
                rows_v.at[buf, pl.ds(c * BW, BW)], gsem,
            ).wait()

    def slab_compute(h, buf):
        rows = rows_v.at[buf]
        outb = outb_v.at[buf]
        lanes = lax.iota(jnp.int32, L)
        for g in range(BW // L):
            bvec = lanes + g * L

            @plsc.parallel_loop(0, EMBED_DIM, 1, unroll=8)
            def _(j):
                k = jnp.bitwise_and(j, L - 1)
                rot = jnp.bitwise_and(lanes + k, L - 1)
                dvec = jnp.bitwise_and(j, 0x30) + rot
                ri = 4 * bvec + lax.shift_right_logical(dvec, 4)
                vals = plsc.load_gather(rows, [ri, jnp.bitwise_and(dvec, 15)])
                dt = lax.shift_right_logical(dvec, 3)
                dl = jnp.bitwise_and(dvec, 7)
                plsc.store_scatter(outb, [dt, dl, bvec], vals * SCALE)

        pltpu.async_copy(
            outb_v.at[buf], out_hbm.at[h, pl.ds(0, 8), wid], osems[buf])

    def wait_out(h, buf):
        pltpu.make_async_copy(
            outb_v.at[buf], out_hbm.at[h, pl.ds(0, 8), wid], osems[buf]
        ).wait()

    for j in range(4):
        start_gather(j, j)

    def body(k, carry):
        h0 = 4 * k
        for buf in range(4):
            h = h0 + buf
            wait_gather(h, buf)

            @pl.when(k > 0)
            def _():
                wait_out(h, buf)

            slab_compute(h, buf)

            @pl.when(k < HIST // 4 - 1)
            def _():
                start_gather(h + 4, buf)

        return carry

    lax.fori_loop(0, HIST // 4, body, 0, unroll=False)
    for j in range(4):
        wait_out(HIST - 4 + j, j)


def kernel(input, table):
    idx4 = jnp.transpose(
        input.reshape(NW, BW, HIST // 8, 8), (2, 0, 3, 1))
    table_rm = _repack(jnp.transpose(table)).reshape(8 * VOCAB, L)
    out5 = _emb_lookup(idx4, table_rm)
    return jnp.transpose(out5, (2, 4, 0, 1, 3)).reshape(BATCH, HIST, EMBED_DIM)
